# baseline (speedup 1.0000x reference)
import numpy as np
import jax
import jax.numpy as jnp
from jax import lax
from jax.experimental import pallas as pl
from jax.experimental.pallas import tpu as pltpu
from jax.experimental.pallas import tpu_sc as plsc

_B, _N = 4096, 128
_M = _N * _N
_K = _N * (_N + 1) // 2
_TK = _K // 8
_NW = 32
_BPW = _B // _NW

_SEG_OFF = [i * (2 * _N + 1 - i) // 2 for i in range(_N + 1)]


def _row_of_k(k):
    import bisect
    return bisect.bisect_right(_SEG_OFF, k) - 1


_TAIL_TILES = []
for _t in (_TK - 2, _TK - 1):
    cols = []
    for _kr in range(8):
        _k = 8 * _t + _kr
        _i = _row_of_k(_k)
        _col = _i + (_k - _SEG_OFF[_i])
        cols.append(8 * (_i - 122) + (_col - 120))
    _TAIL_TILES.append((_t, cols))


def _sc_body(x_hbm, out_hbm, slab_a, slab_b, tail_v, stage_v, isem, osem):
    c = lax.axis_index("c")
    s = lax.axis_index("s")
    w = s * 2 + c
    bc = w * _BPW

    iota = lax.iota(jnp.int32, 16)

    def start_in(slab, m):
        pltpu.async_copy(x_hbm.at[pl.ds(bc, _BPW), pl.ds(m * 256, 256)],
                         slab, isem)

    def wait_in(slab, m):
        pltpu.make_async_copy(
            x_hbm.at[pl.ds(bc, _BPW), pl.ds(m * 256, 256)], slab,
            isem).wait()

    def drain_out():
        pltpu.make_async_copy(stage_v.at[0], out_hbm.at[0, w], osem).wait()

    def ring_pre(g):
        @pl.when(g >= 8)
        def _():
            drain_out()

    def ring_post(t, g):
        pltpu.async_copy(stage_v.at[g & 7], out_hbm.at[t, w], osem)
        return g + 1

    def gather_row(slab, par, kr, colv):
        vals = []
        for bs in range(8):
            br = iota + (bs * 16)
            vals.append(plsc.load_gather(slab, [br, colv]))
        for bs in range(8):
            stage_v[par, kr, pl.ds(bs * 16, 16)] = vals[bs]

    def full_strips(slab, base, i, off_i, off_n, g):
        t_lo = (off_i + 7) // 8
        t_hi = jnp.maximum(t_lo, off_n // 8)

        def body(t, g2):
            ring_pre(g2)
            colbase = jnp.full((16,), base + i + (8 * t - off_i),
                               dtype=jnp.int32)
            for kr in range(8):
                gather_row(slab, g2 & 7, kr, colbase + kr)
            return ring_post(t, g2)

        return lax.fori_loop(t_lo, t_hi, body, g)

    def straddle(prev, pbase0, cur, cbase0, i, off_i, g):
        t0 = off_i // 8
        n1 = off_i - 8 * t0

        def do(g2):
            ring_pre(g2)
            par = g2 & 7
            pbase = jnp.full((16,), pbase0 + _N - n1, dtype=jnp.int32)
            cbase = jnp.full((16,), cbase0 + i - n1, dtype=jnp.int32)

            def kprev(kr, u):
                gather_row(prev, par, kr, pbase + kr)
                return u

            def kcur(kr, u):
                gather_row(cur, par, kr, cbase + kr)
                return u

            lax.fori_loop(0, n1, kprev, 0)
            lax.fori_loop(n1, 8, kcur, 0)
            return ring_post(t0, g2)

        return lax.cond((n1 != 0) & (t0 <= _TK - 3) & (i > 0),
                        do, lambda g2: g2, g)

    start_in(slab_a, 0)

    def quad(q, g):
        i0 = q * 4
        offs = [None] * 5
        offs[0] = i0 * (2 * _N + 1 - i0) // 2
        for j in range(4):
            offs[j + 1] = offs[j] + (_N - (i0 + j))

        wait_in(slab_a, 2 * q)
        g = straddle(slab_b, 128, slab_a, 0, i0, offs[0], g)
        start_in(slab_b, 2 * q + 1)
        g = full_strips(slab_a, 0, i0, offs[0], offs[1], g)
        g = straddle(slab_a, 0, slab_a, 128, i0 + 1, offs[1], g)
        g = full_strips(slab_a, 128, i0 + 1, offs[1], offs[2], g)

        wait_in(slab_b, 2 * q + 1)
        g = straddle(slab_a, 128, slab_b, 0, i0 + 2, offs[2], g)

        @pl.when(i0 + 4 < _N)
        def _():
            start_in(slab_a, 2 * q + 2)

        g = full_strips(slab_b, 0, i0 + 2, offs[2], offs[3], g)
        g = straddle(slab_b, 0, slab_b, 128, i0 + 3, offs[3], g)
        g = full_strips(slab_b, 128, i0 + 3, offs[3], offs[4], g)
        return g

    g = lax.fori_loop(0, _N // 4, quad, 0)

    for j in range(6):
        pltpu.sync_copy(
            x_hbm.at[pl.ds(bc, _BPW), pl.ds((122 + j) * _N + 120, 8)],
            tail_v.at[:, pl.ds(8 * j, 8)])

    for t, cols in _TAIL_TILES:
        ring_pre(g)
        par_t = g & 7
        for kr in range(8):
            colv = jnp.full((16,), cols[kr], dtype=jnp.int32)
            gather_row(tail_v, par_t, kr, colv)
        g = ring_post(t, g)

    for _ in range(8):
        drain_out()


def kernel(input):
    x2 = input.reshape(_B, _M)
    mesh = plsc.VectorSubcoreMesh(core_axis_name="c", subcore_axis_name="s")
    f = pl.kernel(
        _sc_body,
        mesh=mesh,
        out_type=jax.ShapeDtypeStruct((_TK, _NW, 8, _BPW), jnp.float32),
        scratch_types=[
            pltpu.VMEM((_BPW, 256), jnp.float32),
            pltpu.VMEM((_BPW, 256), jnp.float32),
            pltpu.VMEM((_BPW, 48), jnp.float32),
            pltpu.VMEM((8, 8, _BPW), jnp.float32),
            pltpu.SemaphoreType.DMA,
            pltpu.SemaphoreType.DMA,
        ],
        compiler_params=pltpu.CompilerParams(
            use_tc_tiling_on_sc=False, needs_layout_passes=False
        ),
    )
    r4 = f(x2)
    return r4.transpose(1, 3, 0, 2).reshape(_B, _K)


# speedup vs baseline: 1.6669x; 1.6669x over previous
import numpy as np
import jax
import jax.numpy as jnp
from jax import lax
from jax.experimental import pallas as pl
from jax.experimental.pallas import tpu as pltpu
from jax.experimental.pallas import tpu_sc as plsc

_B, _N = 4096, 128
_M = _N * _N
_K = _N * (_N + 1) // 2
_NT = _K // 16

_NW = 32
_BPW = _B // _NW

_ROW_IDX, _COL_IDX = np.triu_indices(_N)
_FLAT_IDX = (_ROW_IDX * _N + _COL_IDX).astype(np.int32)

_SEG_OFF = np.concatenate([[0], np.cumsum(np.arange(_N, 0, -1))])
_TILE_PLAN = []
for _t in range(_NT):
    _lo = 16 * _t
    _i = int(np.searchsorted(_SEG_OFF, _lo, side="right") - 1)
    if _SEG_OFF[_i + 1] >= _lo + 16:
        _TILE_PLAN.append((_lo, _i * (_N + 1) + (_lo - int(_SEG_OFF[_i]))))
    else:
        _TILE_PLAN.append((_lo, None))


def _sc_body(x_hbm, idx_hbm, out_hbm, idx_v, in0, in1, ou0, ou1,
             is0, is1, os0, os1):
    c = lax.axis_index("c")
    s = lax.axis_index("s")
    wid = s * 2 + c
    b0 = wid * _BPW

    pltpu.sync_copy(idx_hbm, idx_v)

    bufs = ((in0, ou0, is0, os0), (in1, ou1, is1, os1))

    def start_in(p, b):
        iv, _, isem, _ = bufs[p]
        pltpu.async_copy(x_hbm.at[b], iv, isem)

    def wait_in(p, b):
        iv, _, isem, _ = bufs[p]
        pltpu.make_async_copy(x_hbm.at[b], iv, isem).wait()

    def start_out(p, b):
        _, ov, _, osem = bufs[p]
        pltpu.async_copy(ov, out_hbm.at[b], osem)

    def wait_out(p, b):
        _, ov, _, osem = bufs[p]
        pltpu.make_async_copy(ov, out_hbm.at[b], osem).wait()

    start_in(0, b0)
    start_in(1, b0 + 1)

    def pair(rr, carry):
        for p in (0, 1):
            r = rr * 2 + p
            b = b0 + r
            iv, ov, _, _ = bufs[p]
            wait_in(p, b)

            @pl.when(rr > 0)
            def _():
                wait_out(p, b - 2)

            for o, so in _TILE_PLAN:
                if so is not None:
                    ov[pl.ds(o, 16)] = iv[pl.ds(so, 16)]
                else:
                    idx = idx_v[pl.ds(o, 16)]
                    ov[pl.ds(o, 16)] = plsc.load_gather(iv, [idx])
            start_out(p, b)

            @pl.when(r + 2 < _BPW)
            def _():
                start_in(p, b + 2)
        return carry

    lax.fori_loop(0, _BPW // 2, pair, 0)

    wait_out(0, b0 + _BPW - 2)
    wait_out(1, b0 + _BPW - 1)


def kernel(input):
    x2 = input.reshape(_B, _M)
    fidx = jnp.asarray(_FLAT_IDX)
    mesh = plsc.VectorSubcoreMesh(core_axis_name="c", subcore_axis_name="s")
    f = pl.kernel(
        _sc_body,
        mesh=mesh,
        out_type=jax.ShapeDtypeStruct((_B, _K), jnp.float32),
        scratch_types=[
            pltpu.VMEM((_K,), jnp.int32),
            pltpu.VMEM((_M,), jnp.float32),
            pltpu.VMEM((_M,), jnp.float32),
            pltpu.VMEM((_K,), jnp.float32),
            pltpu.VMEM((_K,), jnp.float32),
            pltpu.SemaphoreType.DMA,
            pltpu.SemaphoreType.DMA,
            pltpu.SemaphoreType.DMA,
            pltpu.SemaphoreType.DMA,
        ],
        compiler_params=pltpu.CompilerParams(
            use_tc_tiling_on_sc=False, needs_layout_passes=False
        ),
    )
    return f(x2, fidx)
